# baseline (speedup 1.0000x reference)
import functools

import jax
import jax.numpy as jnp
from jax import lax
from jax.experimental import pallas as pl
from jax.experimental.pallas import tpu as pltpu
from jax.experimental.pallas import tpu_sc as plsc

DICT_SIZE = 1000
_NC = 2
_NS = 16
_R = 32


def _sc_onehot_body(n_per_w, idx_hbm, out_hbm, idx_v, buf0, buf1, sem0, sem1):
    D = DICT_SIZE
    wid = lax.axis_index("s") * _NC + lax.axis_index("c")
    base = wid * n_per_w
    bufs = (buf0, buf1)
    sems = (sem0, sem1)

    pltpu.sync_copy(idx_hbm.at[pl.ds(base * 1, n_per_w)], idx_v)

    zeros16 = jnp.zeros((16,), jnp.float32)
    ones16 = jnp.ones((16,), jnp.float32)
    lane = lax.iota(jnp.int32, 16)

    def _zero(i, carry):
        buf0[pl.ds(i * 16, 16)] = zeros16
        buf1[pl.ds(i * 16, 16)] = zeros16
        return carry

    lax.fori_loop(0, (_R * D) // 16, _zero, 0, unroll=8)

    nb = n_per_w // _R

    def _scatter(bi, slot, val16):
        row0 = bi * _R
        for ck in range(_R // 16):
            idxs = idx_v[pl.ds(row0 + ck * 16, 16)]
            offs = (lane + ck * 16) * D + idxs
            plsc.store_scatter(bufs[slot], [offs], val16)

    def _issue(bi, slot):
        pltpu.async_copy(
            bufs[slot], out_hbm.at[pl.ds((base + bi * _R) * D, _R * D)], sems[slot]
        )

    def _wait(bi, slot):
        pltpu.make_async_copy(
            bufs[slot], out_hbm.at[pl.ds((base + bi * _R) * D, _R * D)], sems[slot]
        ).wait()

    def _pair(g, carry):
        for b in range(2):
            bi = 2 * g + b

            @pl.when(g >= 1)
            def _():
                _wait(bi - 2, b)
                _scatter(bi - 2, b, zeros16)

            _scatter(bi, b, ones16)
            _issue(bi, b)
        return carry

    lax.fori_loop(0, nb // 2, _pair, 0)
    _wait(nb - 2, 0)
    _wait(nb - 1, 1)


def kernel(input, teacher_forcing):
    S, B = input.shape
    N = S * B
    n_per_w = N // (_NC * _NS)
    flat_idx = input.reshape(N).astype(jnp.int32)

    sc_call = pl.kernel(
        functools.partial(_sc_onehot_body, n_per_w),
        out_type=jax.ShapeDtypeStruct((N * DICT_SIZE,), jnp.float32),
        mesh=plsc.VectorSubcoreMesh(core_axis_name="c", subcore_axis_name="s"),
        scratch_types=[
            pltpu.VMEM((n_per_w,), jnp.int32),
            pltpu.VMEM((_R * DICT_SIZE,), jnp.float32),
            pltpu.VMEM((_R * DICT_SIZE,), jnp.float32),
            pltpu.SemaphoreType.DMA,
            pltpu.SemaphoreType.DMA,
        ],
        compiler_params=pltpu.CompilerParams(needs_layout_passes=False),
    )
    p2 = sc_call(flat_idx).reshape(S, B, DICT_SIZE)
    return (p2, input)


# speedup vs baseline: 1.1683x; 1.1683x over previous
import functools

import jax
import jax.numpy as jnp
from jax import lax
from jax.experimental import pallas as pl
from jax.experimental.pallas import tpu as pltpu
from jax.experimental.pallas import tpu_sc as plsc

DICT_SIZE = 1000
_NC = 2
_NS = 16
_R = 64


def _sc_onehot_body(n_per_w, idx_hbm, out_hbm, idx_v, buf0, buf1, sem0, sem1):
    D = DICT_SIZE
    wid = lax.axis_index("s") * _NC + lax.axis_index("c")
    base = wid * n_per_w
    bufs = (buf0, buf1)
    sems = (sem0, sem1)

    pltpu.sync_copy(idx_hbm.at[pl.ds(base * 1, n_per_w)], idx_v)

    zeros16 = jnp.zeros((16,), jnp.float32)
    ones16 = jnp.ones((16,), jnp.float32)
    lane = lax.iota(jnp.int32, 16)

    def _zero(i, carry):
        buf0[pl.ds(i * 16, 16)] = zeros16
        buf1[pl.ds(i * 16, 16)] = zeros16
        return carry

    lax.fori_loop(0, (_R * D) // 16, _zero, 0, unroll=8)

    nb = n_per_w // _R

    def _scatter(bi, slot, val16):
        row0 = bi * _R
        for ck in range(_R // 16):
            idxs = idx_v[pl.ds(row0 + ck * 16, 16)]
            offs = (lane + ck * 16) * D + idxs
            plsc.store_scatter(bufs[slot], [offs], val16)

    def _issue(bi, slot):
        pltpu.async_copy(
            bufs[slot], out_hbm.at[pl.ds((base + bi * _R) * D, _R * D)], sems[slot]
        )

    def _wait(bi, slot):
        pltpu.make_async_copy(
            bufs[slot], out_hbm.at[pl.ds((base + bi * _R) * D, _R * D)], sems[slot]
        ).wait()

    def _pair(g, carry):
        for b in range(2):
            bi = 2 * g + b

            @pl.when(jnp.logical_and(g >= 1, bi < nb))
            def _():
                _wait(bi - 2, b)
                _scatter(bi - 2, b, zeros16)

            @pl.when(bi < nb)
            def _():
                _scatter(bi, b, ones16)
                _issue(bi, b)
        return carry

    lax.fori_loop(0, (nb + 1) // 2, _pair, 0)
    _wait(nb - 2, (nb - 2) % 2)
    _wait(nb - 1, (nb - 1) % 2)


def kernel(input, teacher_forcing):
    S, B = input.shape
    N = S * B
    n_per_w = N // (_NC * _NS)
    flat_idx = input.reshape(N).astype(jnp.int32)

    sc_call = pl.kernel(
        functools.partial(_sc_onehot_body, n_per_w),
        out_type=jax.ShapeDtypeStruct((N * DICT_SIZE,), jnp.float32),
        mesh=plsc.VectorSubcoreMesh(core_axis_name="c", subcore_axis_name="s"),
        scratch_types=[
            pltpu.VMEM((n_per_w,), jnp.int32),
            pltpu.VMEM((_R * DICT_SIZE,), jnp.float32),
            pltpu.VMEM((_R * DICT_SIZE,), jnp.float32),
            pltpu.SemaphoreType.DMA,
            pltpu.SemaphoreType.DMA,
        ],
        compiler_params=pltpu.CompilerParams(needs_layout_passes=False),
    )
    p2 = sc_call(flat_idx).reshape(S, B, DICT_SIZE)
    return (p2, input)


# speedup vs baseline: 2.2846x; 1.9555x over previous
import functools

import jax
import jax.numpy as jnp
from jax import lax
from jax.experimental import pallas as pl
from jax.experimental.pallas import tpu as pltpu
from jax.experimental.pallas import tpu_sc as plsc

DICT_SIZE = 1000
_NC = 2
_NS = 16
_R = 64


def _sc_onehot_body(n_per_w, idx_hbm, out_hbm, idx_v, buf, sem):
    D = DICT_SIZE
    wid = lax.axis_index("s") * _NC + lax.axis_index("c")
    base = wid * n_per_w

    pltpu.sync_copy(idx_hbm.at[pl.ds(base * 1, n_per_w)], idx_v)

    zeros16 = jnp.zeros((16,), jnp.float32)
    ones16 = jnp.ones((16,), jnp.float32)
    lane = lax.iota(jnp.int32, 16)
    nfull = D // 16
    tail = D - nfull * 16
    tail_mask = lane < tail

    def _zero_row(r, carry):
        for c in range(nfull):
            buf[r, pl.ds(c * 16, 16)] = zeros16
        plsc.store_scatter(
            buf, [jnp.full((16,), r, jnp.int32), nfull * 16 + lane],
            zeros16, mask=tail_mask,
        )
        return carry

    lax.fori_loop(0, _R, _zero_row, 0)

    nb = n_per_w // _R

    def _scatter(bi, val16):
        row0 = bi * _R
        for ck in range(_R // 16):
            idxs = idx_v[pl.ds(row0 + ck * 16, 16)]
            plsc.store_scatter(buf, [lane + ck * 16, idxs], val16)

    def _batch(bi, carry):
        _scatter(bi, ones16)
        pltpu.sync_copy(buf, out_hbm.at[pl.ds(base + bi * _R, _R), :])
        _scatter(bi, zeros16)
        return carry

    lax.fori_loop(0, nb, _batch, 0)


def kernel(input, teacher_forcing):
    S, B = input.shape
    N = S * B
    n_per_w = N // (_NC * _NS)
    flat_idx = input.reshape(N).astype(jnp.int32)

    sc_call = pl.kernel(
        functools.partial(_sc_onehot_body, n_per_w),
        out_type=jax.ShapeDtypeStruct((N, DICT_SIZE), jnp.float32),
        mesh=plsc.VectorSubcoreMesh(core_axis_name="c", subcore_axis_name="s"),
        scratch_types=[
            pltpu.VMEM((n_per_w,), jnp.int32),
            pltpu.VMEM((_R, DICT_SIZE), jnp.float32),
            pltpu.SemaphoreType.DMA,
        ],
        compiler_params=pltpu.CompilerParams(needs_layout_passes=False),
    )
    p2 = sc_call(flat_idx).reshape(S, B, DICT_SIZE)
    return (p2, input)
